# baseline (speedup 1.0000x reference)
import functools

import jax
import jax.numpy as jnp
from jax import lax
from jax.experimental import pallas as pl
from jax.experimental.pallas import tpu as pltpu
from jax.experimental.pallas import tpu_sc as plsc

NUM_CORES = 2
NUM_SUBCORES = 16
NUM_WORKERS = NUM_CORES * NUM_SUBCORES
LANES = 16
BCHUNK = 128
NT = 3


def _make_kernel(B, L, D, V):
    n_bblk = B // BCHUNK
    n_chunks = L * n_bblk
    n_chunks_w = n_chunks // NUM_WORKERS
    DM = NT * D
    mesh = plsc.VectorSubcoreMesh(core_axis_name="c", subcore_axis_name="s")

    @functools.partial(
        pl.kernel,
        mesh=mesh,
        compiler_params=pltpu.CompilerParams(use_tc_tiling_on_sc=False,
                                             needs_layout_passes=False),
        out_type=jax.ShapeDtypeStruct((L, DM // 8, B // BCHUNK, 8, BCHUNK),
                                      jnp.float32),
        scratch_types=[
            pltpu.VMEM((NT, BCHUNK), jnp.int32),
            pltpu.VMEM((NT, BCHUNK), jnp.int32),
            pltpu.VMEM((NT, BCHUNK, D), jnp.float32),
            pltpu.VMEM((NT, BCHUNK, D), jnp.float32),
            pltpu.VMEM((DM // 8, 8, BCHUNK), jnp.float32),
            pltpu.VMEM((DM // 8, 8, BCHUNK), jnp.float32),
            pltpu.VMEM((L, DM), jnp.float32),
            pltpu.SemaphoreType.DMA,
            pltpu.SemaphoreType.DMA,
            pltpu.SemaphoreType.DMA,
            pltpu.SemaphoreType.DMA,
        ],
    )
    def emb_kernel(idxB, t0, t1, t2, pos, out,
                   iv0, iv1, r0, r1, ct0, ct1, pv, sg0, sg1, sw0, sw1):
        wid = lax.axis_index("s") * NUM_CORES + lax.axis_index("c")
        tables = (t0, t1, t2)
        lane = lax.iota(jnp.int32, LANES)

        pltpu.sync_copy(pos.at[pl.ds(0, L), pl.ds(0, DM)], pv)

        def coords(c):
            gc = wid * n_chunks_w + c
            return gc // n_bblk, gc % n_bblk

        def load_idx(c, iv):
            l, cg = coords(c)
            for j in range(NT):
                pltpu.sync_copy(idxB.at[j, l, pl.ds(cg * BCHUNK, BCHUNK)],
                                iv.at[j])

        def fire_gathers(iv, r, sem):
            for j in range(NT):
                pltpu.async_copy(tables[j].at[iv.at[j]], r.at[j], sem)

        def wait_gathers(iv, r, sem):
            for j in range(NT):
                pltpu.make_async_copy(tables[j].at[iv.at[j]],
                                      r.at[j], sem).wait()

        def add_transpose(c, r, ct):
            l, _ = coords(c)
            pvecs = [pv[l, pl.ds(h * LANES, LANES)] for h in range(DM // LANES)]
            dl = [(h * LANES) + lane for h in range(DM // LANES)]
            rgv = [lax.shift_right_logical(d, 1 + 1 + 1) for d in dl]
            rv = [lax.bitwise_and(d, 7) for d in dl]

            def tok_body(t, carry):
                bvec = lane * 0 + t
                for j in range(NT):
                    for hh in range(D // LANES):
                        h = j * (D // LANES) + hh
                        x = r[j, t, pl.ds(hh * LANES, LANES)] + pvecs[h]
                        plsc.store_scatter(ct, [rgv[h], rv[h], bvec], x)
                return carry

            lax.fori_loop(0, BCHUNK, tok_body, 0)

        def start_write(c, ct, sem):
            l, cg = coords(c)
            return pltpu.async_copy(ct, out.at[l, :, cg], sem)

        def wait_write(c, ct, sem):
            l, cg = coords(c)
            pltpu.make_async_copy(ct, out.at[l, :, cg], sem).wait()

        bufs = ((iv0, r0, ct0, sg0, sw0), (iv1, r1, ct1, sg1, sw1))

        load_idx(0, iv0)
        fire_gathers(iv0, r0, sg0)
        load_idx(1, iv1)

        def pair_body(i, carry):
            for half in range(2):
                c = 2 * i + half
                iv_c, r_c, ct_c, sg_c, sw_c = bufs[half]
                iv_n, r_n, ct_n, sg_n, sw_n = bufs[1 - half]

                @pl.when(c + 1 < n_chunks_w)
                def _():
                    fire_gathers(iv_n, r_n, sg_n)

                wait_gathers(iv_c, r_c, sg_c)

                @pl.when(c + 2 < n_chunks_w)
                def _():
                    load_idx(c + 2, iv_c)

                @pl.when(c >= 2)
                def _():
                    wait_write(c - 2, ct_c, sw_c)

                add_transpose(c, r_c, ct_c)
                start_write(c, ct_c, sw_c)
            return carry

        lax.fori_loop(0, n_chunks_w // 2, pair_body, 0)

        wait_write(n_chunks_w - 2, ct0, sw0)
        wait_write(n_chunks_w - 1, ct1, sw1)

    return emb_kernel


def kernel(batch, W_opcode, W_operand1, W_operand2, W_pos):
    B, L, nt = batch.shape
    V, D = W_opcode.shape
    assert nt == NT and B % BCHUNK == 0
    assert (L * (B // BCHUNK)) % (2 * NUM_WORKERS) == 0

    idxB = batch.astype(jnp.int32).transpose(2, 1, 0)
    out = _make_kernel(B, L, D, V)(
        idxB, W_opcode, W_operand1, W_operand2, W_pos)
    return out.transpose(2, 4, 0, 1, 3).reshape(B, L, NT * D)


# speedup vs baseline: 1.3181x; 1.3181x over previous
import functools

import jax
import jax.numpy as jnp
from jax import lax
from jax.experimental import pallas as pl
from jax.experimental.pallas import tpu as pltpu
from jax.experimental.pallas import tpu_sc as plsc

NUM_CORES = 2
NUM_SUBCORES = 16
NUM_WORKERS = NUM_CORES * NUM_SUBCORES
LANES = 16
BCHUNK = 128
NT = 3


def _make_kernel(B, L, D, V):
    n_bblk = B // BCHUNK
    n_chunks = L * n_bblk
    n_chunks_w = n_chunks // NUM_WORKERS
    DM = NT * D
    mesh = plsc.VectorSubcoreMesh(core_axis_name="c", subcore_axis_name="s")

    @functools.partial(
        pl.kernel,
        mesh=mesh,
        compiler_params=pltpu.CompilerParams(use_tc_tiling_on_sc=False,
                                             needs_layout_passes=False),
        out_type=jax.ShapeDtypeStruct((L, DM // 8, B // BCHUNK, 8, BCHUNK),
                                      jnp.float32),
        scratch_types=[
            pltpu.VMEM((NT, BCHUNK), jnp.int32),
            pltpu.VMEM((NT, BCHUNK), jnp.int32),
            pltpu.VMEM((NT, BCHUNK, D), jnp.float32),
            pltpu.VMEM((NT, BCHUNK, D), jnp.float32),
            pltpu.VMEM((DM // 8, 8, BCHUNK), jnp.float32),
            pltpu.VMEM((DM // 8, 8, BCHUNK), jnp.float32),
            pltpu.VMEM((L, DM), jnp.float32),
            pltpu.SemaphoreType.DMA,
            pltpu.SemaphoreType.DMA,
            pltpu.SemaphoreType.DMA,
            pltpu.SemaphoreType.DMA,
        ],
    )
    def emb_kernel(idxB, t0, t1, t2, pos, out,
                   iv0, iv1, r0, r1, ct0, ct1, pv, sg0, sg1, sw0, sw1):
        wid = lax.axis_index("s") * NUM_CORES + lax.axis_index("c")
        tables = (t0, t1, t2)
        lane = lax.iota(jnp.int32, LANES)

        pltpu.sync_copy(pos.at[pl.ds(0, L), pl.ds(0, DM)], pv)

        def coords(c):
            gc = wid * n_chunks_w + c
            return gc // n_bblk, gc % n_bblk

        def load_idx(c, iv):
            l, cg = coords(c)
            for j in range(NT):
                pltpu.sync_copy(idxB.at[j, l, pl.ds(cg * BCHUNK, BCHUNK)],
                                iv.at[j])

        def fire_gathers(iv, r, sem):
            for j in range(NT):
                pltpu.async_copy(tables[j].at[iv.at[j]], r.at[j], sem)

        def wait_gathers(iv, r, sem):
            for j in range(NT):
                pltpu.make_async_copy(tables[j].at[iv.at[j]],
                                      r.at[j], sem).wait()

        def add_transpose(c, r, ct):
            l, _ = coords(c)
            pvecs = [pv[l, pl.ds(h * LANES, LANES)] for h in range(DM // LANES)]
            dl = [(h * LANES) + lane for h in range(DM // LANES)]
            rgv = [lax.shift_right_logical(d, 1 + 1 + 1) for d in dl]
            rv = [lax.bitwise_and(d, 7) for d in dl]

            @plsc.parallel_loop(0, BCHUNK, unroll=2)
            def tok_body(t):
                bvec = lane * 0 + t
                for j in range(NT):
                    for hh in range(D // LANES):
                        h = j * (D // LANES) + hh
                        x = r[j, t, pl.ds(hh * LANES, LANES)] + pvecs[h]
                        plsc.store_scatter(ct, [rgv[h], rv[h], bvec], x)

        def start_write(c, ct, sem):
            l, cg = coords(c)
            return pltpu.async_copy(ct, out.at[l, :, cg], sem)

        def wait_write(c, ct, sem):
            l, cg = coords(c)
            pltpu.make_async_copy(ct, out.at[l, :, cg], sem).wait()

        bufs = ((iv0, r0, ct0, sg0, sw0), (iv1, r1, ct1, sg1, sw1))

        load_idx(0, iv0)
        fire_gathers(iv0, r0, sg0)
        load_idx(1, iv1)

        def pair_body(i, carry):
            for half in range(2):
                c = 2 * i + half
                iv_c, r_c, ct_c, sg_c, sw_c = bufs[half]
                iv_n, r_n, ct_n, sg_n, sw_n = bufs[1 - half]

                @pl.when(c + 1 < n_chunks_w)
                def _():
                    fire_gathers(iv_n, r_n, sg_n)

                wait_gathers(iv_c, r_c, sg_c)

                @pl.when(c + 2 < n_chunks_w)
                def _():
                    load_idx(c + 2, iv_c)

                @pl.when(c >= 2)
                def _():
                    wait_write(c - 2, ct_c, sw_c)

                add_transpose(c, r_c, ct_c)
                start_write(c, ct_c, sw_c)
            return carry

        lax.fori_loop(0, n_chunks_w // 2, pair_body, 0)

        wait_write(n_chunks_w - 2, ct0, sw0)
        wait_write(n_chunks_w - 1, ct1, sw1)

    return emb_kernel


def kernel(batch, W_opcode, W_operand1, W_operand2, W_pos):
    B, L, nt = batch.shape
    V, D = W_opcode.shape
    assert nt == NT and B % BCHUNK == 0
    assert (L * (B // BCHUNK)) % (2 * NUM_WORKERS) == 0

    idxB = batch.astype(jnp.int32).transpose(2, 1, 0)
    out = _make_kernel(B, L, D, V)(
        idxB, W_opcode, W_operand1, W_operand2, W_pos)
    return out.transpose(2, 4, 0, 1, 3).reshape(B, L, NT * D)


# speedup vs baseline: 1.3185x; 1.0003x over previous
import functools

import jax
import jax.numpy as jnp
from jax import lax
from jax.experimental import pallas as pl
from jax.experimental.pallas import tpu as pltpu
from jax.experimental.pallas import tpu_sc as plsc

NUM_CORES = 2
NUM_SUBCORES = 16
NUM_WORKERS = NUM_CORES * NUM_SUBCORES
LANES = 16
BCHUNK = 128
NT = 3


def _make_kernel(B, L, D, V):
    n_bblk = B // BCHUNK
    n_chunks = L * n_bblk
    n_chunks_w = n_chunks // NUM_WORKERS
    DM = NT * D
    mesh = plsc.VectorSubcoreMesh(core_axis_name="c", subcore_axis_name="s")

    @functools.partial(
        pl.kernel,
        mesh=mesh,
        compiler_params=pltpu.CompilerParams(use_tc_tiling_on_sc=False,
                                             needs_layout_passes=False),
        out_type=jax.ShapeDtypeStruct((L, DM // 8, B // BCHUNK, 8, BCHUNK),
                                      jnp.float32),
        scratch_types=[
            pltpu.VMEM((NT, BCHUNK), jnp.int32),
            pltpu.VMEM((NT, BCHUNK), jnp.int32),
            pltpu.VMEM((NT, BCHUNK, D), jnp.float32),
            pltpu.VMEM((NT, BCHUNK, D), jnp.float32),
            pltpu.VMEM((DM // 8, 8, BCHUNK), jnp.float32),
            pltpu.VMEM((DM // 8, 8, BCHUNK), jnp.float32),
            pltpu.VMEM((L, DM), jnp.float32),
            pltpu.SemaphoreType.DMA,
            pltpu.SemaphoreType.DMA,
            pltpu.SemaphoreType.DMA,
            pltpu.SemaphoreType.DMA,
        ],
    )
    def emb_kernel(idxB, t0, t1, t2, pos, out,
                   iv0, iv1, r0, r1, ct0, ct1, pv, sg0, sg1, sw0, sw1):
        wid = lax.axis_index("s") * NUM_CORES + lax.axis_index("c")
        tables = (t0, t1, t2)
        lane = lax.iota(jnp.int32, LANES)

        pltpu.sync_copy(pos.at[pl.ds(0, L), pl.ds(0, DM)], pv)

        def coords(c):
            gc = wid * n_chunks_w + c
            return gc // n_bblk, gc % n_bblk

        def load_idx(c, iv):
            l, cg = coords(c)
            for j in range(NT):
                pltpu.sync_copy(idxB.at[j, l, pl.ds(cg * BCHUNK, BCHUNK)],
                                iv.at[j])

        def fire_gathers(iv, r, sem):
            for j in range(NT):
                pltpu.async_copy(tables[j].at[iv.at[j]], r.at[j], sem)

        def wait_gathers(iv, r, sem):
            for j in range(NT):
                pltpu.make_async_copy(tables[j].at[iv.at[j]],
                                      r.at[j], sem).wait()

        def add_transpose(c, r, ct):
            l, _ = coords(c)
            pvecs = [pv[l, pl.ds(h * LANES, LANES)] for h in range(DM // LANES)]
            dl = [(h * LANES) + lane for h in range(DM // LANES)]
            rgv = [lax.shift_right_logical(d, 1 + 1 + 1) for d in dl]
            rv = [lax.bitwise_and(d, 7) for d in dl]

            @plsc.parallel_loop(0, BCHUNK, unroll=4)
            def tok_body(t):
                bvec = lane * 0 + t
                for j in range(NT):
                    for hh in range(D // LANES):
                        h = j * (D // LANES) + hh
                        x = r[j, t, pl.ds(hh * LANES, LANES)] + pvecs[h]
                        plsc.store_scatter(ct, [rgv[h], rv[h], bvec], x)

        def start_write(c, ct, sem):
            l, cg = coords(c)
            return pltpu.async_copy(ct, out.at[l, :, cg], sem)

        def wait_write(c, ct, sem):
            l, cg = coords(c)
            pltpu.make_async_copy(ct, out.at[l, :, cg], sem).wait()

        bufs = ((iv0, r0, ct0, sg0, sw0), (iv1, r1, ct1, sg1, sw1))

        load_idx(0, iv0)
        fire_gathers(iv0, r0, sg0)
        load_idx(1, iv1)

        def pair_body(i, carry):
            for half in range(2):
                c = 2 * i + half
                iv_c, r_c, ct_c, sg_c, sw_c = bufs[half]
                iv_n, r_n, ct_n, sg_n, sw_n = bufs[1 - half]

                @pl.when(c + 1 < n_chunks_w)
                def _():
                    fire_gathers(iv_n, r_n, sg_n)

                wait_gathers(iv_c, r_c, sg_c)

                @pl.when(c + 2 < n_chunks_w)
                def _():
                    load_idx(c + 2, iv_c)

                @pl.when(c >= 2)
                def _():
                    wait_write(c - 2, ct_c, sw_c)

                add_transpose(c, r_c, ct_c)
                start_write(c, ct_c, sw_c)
            return carry

        lax.fori_loop(0, n_chunks_w // 2, pair_body, 0)

        wait_write(n_chunks_w - 2, ct0, sw0)
        wait_write(n_chunks_w - 1, ct1, sw1)

    return emb_kernel


def kernel(batch, W_opcode, W_operand1, W_operand2, W_pos):
    B, L, nt = batch.shape
    V, D = W_opcode.shape
    assert nt == NT and B % BCHUNK == 0
    assert (L * (B // BCHUNK)) % (2 * NUM_WORKERS) == 0

    idxB = batch.astype(jnp.int32).transpose(2, 1, 0)
    out = _make_kernel(B, L, D, V)(
        idxB, W_opcode, W_operand1, W_operand2, W_pos)
    return out.transpose(2, 4, 0, 1, 3).reshape(B, L, NT * D)


# speedup vs baseline: 2.3810x; 1.8059x over previous
import functools

import jax
import jax.numpy as jnp
from jax import lax
from jax.experimental import pallas as pl
from jax.experimental.pallas import tpu as pltpu
from jax.experimental.pallas import tpu_sc as plsc

NUM_CORES = 2
NUM_SUBCORES = 16
NUM_WORKERS = NUM_CORES * NUM_SUBCORES
LANES = 16
BCHUNK = 128
NT = 3


def _make_kernel(B, L, D, V):
    n_bblk = B // BCHUNK
    n_chunks = L * n_bblk
    n_chunks_w = n_chunks // NUM_WORKERS
    DM = NT * D
    mesh = plsc.VectorSubcoreMesh(core_axis_name="c", subcore_axis_name="s")

    @functools.partial(
        pl.kernel,
        mesh=mesh,
        compiler_params=pltpu.CompilerParams(use_tc_tiling_on_sc=False,
                                             needs_layout_passes=False),
        out_type=jax.ShapeDtypeStruct((L, DM // 8, B // BCHUNK, 8, BCHUNK),
                                      jnp.float32),
        scratch_types=[
            pltpu.VMEM((NT, BCHUNK), jnp.int32),
            pltpu.VMEM((NT, BCHUNK), jnp.int32),
            pltpu.VMEM((NT, BCHUNK), jnp.int32),
            pltpu.VMEM((NT, BCHUNK, D), jnp.float32),
            pltpu.VMEM((NT, BCHUNK, D), jnp.float32),
            pltpu.VMEM((NT, BCHUNK, D), jnp.float32),
            pltpu.VMEM((DM // 8, 8, BCHUNK + 1), jnp.float32),
            pltpu.VMEM((DM // 8, 8, BCHUNK + 1), jnp.float32),
            pltpu.VMEM((DM // 8, 8, BCHUNK + 1), jnp.float32),
            pltpu.VMEM((L, DM), jnp.float32),
            pltpu.SemaphoreType.DMA,
            pltpu.SemaphoreType.DMA,
            pltpu.SemaphoreType.DMA,
            pltpu.SemaphoreType.DMA,
            pltpu.SemaphoreType.DMA,
            pltpu.SemaphoreType.DMA,
        ],
    )
    def emb_kernel(idxB, t0, t1, t2, pos, out,
                   iv0, iv1, iv2, r0, r1, r2, ct0, ct1, ct2, pv,
                   sg0, sg1, sg2, sw0, sw1, sw2):
        wid = lax.axis_index("s") * NUM_CORES + lax.axis_index("c")
        tables = (t0, t1, t2)
        lane = lax.iota(jnp.int32, LANES)

        pltpu.sync_copy(pos.at[pl.ds(0, L), pl.ds(0, DM)], pv)

        def coords(c):
            gc = wid * n_chunks_w + c
            return gc // n_bblk, gc % n_bblk

        def load_idx(c, iv):
            l, cg = coords(c)
            for j in range(NT):
                pltpu.sync_copy(idxB.at[j, l, pl.ds(cg * BCHUNK, BCHUNK)],
                                iv.at[j])

        def fire_gathers(iv, r, sem):
            for j in range(NT):
                pltpu.async_copy(tables[j].at[iv.at[j]], r.at[j], sem)

        def wait_gathers(iv, r, sem):
            for j in range(NT):
                pltpu.make_async_copy(tables[j].at[iv.at[j]],
                                      r.at[j], sem).wait()

        def add_transpose(c, r, ct):
            l, _ = coords(c)
            pvecs = [pv[l, pl.ds(h * LANES, LANES)] for h in range(DM // LANES)]
            dl = [(h * LANES) + lane for h in range(DM // LANES)]
            rgv = [lax.shift_right_logical(d, 1 + 1 + 1) for d in dl]
            rv = [lax.bitwise_and(d, 7) for d in dl]

            @plsc.parallel_loop(0, BCHUNK, unroll=4)
            def tok_body(t):
                bvec = lane * 0 + t
                for j in range(NT):
                    for hh in range(D // LANES):
                        h = j * (D // LANES) + hh
                        x = r[j, t, pl.ds(hh * LANES, LANES)] + pvecs[h]
                        plsc.store_scatter(ct, [rgv[h], rv[h], bvec], x)

        def start_write(c, ct, sem):
            l, cg = coords(c)
            return pltpu.async_copy(ct.at[:, :, pl.ds(0, BCHUNK)],
                                    out.at[l, :, cg], sem)

        def wait_write(c, ct, sem):
            l, cg = coords(c)
            pltpu.make_async_copy(ct.at[:, :, pl.ds(0, BCHUNK)],
                                  out.at[l, :, cg], sem).wait()

        bufs = ((iv0, r0, ct0, sg0, sw0), (iv1, r1, ct1, sg1, sw1),
                (iv2, r2, ct2, sg2, sw2))

        load_idx(0, iv0)
        fire_gathers(iv0, r0, sg0)
        load_idx(1, iv1)
        fire_gathers(iv1, r1, sg1)
        load_idx(2, iv2)

        def triple_body(i, carry):
            for ph in range(3):
                c = 3 * i + ph
                iv_c, r_c, ct_c, sg_c, sw_c = bufs[ph]
                iv_n, r_n, ct_n, sg_n, sw_n = bufs[(ph + 2) % 3]

                @pl.when(c + 2 < n_chunks_w)
                def _():
                    fire_gathers(iv_n, r_n, sg_n)

                wait_gathers(iv_c, r_c, sg_c)

                @pl.when(c + 3 < n_chunks_w)
                def _():
                    load_idx(c + 3, iv_c)

                @pl.when(c >= 3)
                def _():
                    wait_write(c - 3, ct_c, sw_c)

                add_transpose(c, r_c, ct_c)
                start_write(c, ct_c, sw_c)
            return carry

        lax.fori_loop(0, n_chunks_w // 3, triple_body, 0)

        for c in range(3 * (n_chunks_w // 3), n_chunks_w):
            iv_c, r_c, ct_c, sg_c, sw_c = bufs[c % 3]
            wait_gathers(iv_c, r_c, sg_c)
            wait_write(c - 3, ct_c, sw_c)
            add_transpose(c, r_c, ct_c)
            start_write(c, ct_c, sw_c)
        for c in range(n_chunks_w - 3, n_chunks_w):
            _, _, ct_c, _, sw_c = bufs[c % 3]
            wait_write(c, ct_c, sw_c)

    return emb_kernel


def kernel(batch, W_opcode, W_operand1, W_operand2, W_pos):
    B, L, nt = batch.shape
    V, D = W_opcode.shape
    assert nt == NT and B % BCHUNK == 0
    assert (L * (B // BCHUNK)) % (2 * NUM_WORKERS) == 0

    idxB = batch.astype(jnp.int32).transpose(2, 1, 0)
    out = _make_kernel(B, L, D, V)(
        idxB, W_opcode, W_operand1, W_operand2, W_pos)
    return out.transpose(2, 4, 0, 1, 3).reshape(B, L, NT * D)
